# baseline (speedup 1.0000x reference)
import jax
import jax.numpy as jnp
from jax.experimental import pallas as pl
from jax.experimental.pallas import tpu as pltpu


_BS = 512


def _body(x_ref, t_ref, o_ref):
    o_ref[...] = x_ref[...] + t_ref[...][None, :, :]


def kernel(x, pos_table):
    B, S, D = x.shape
    bs = _BS if S % _BS == 0 else S
    return pl.pallas_call(
        _body,
        grid=(S // bs,),
        in_specs=[
            pl.BlockSpec((B, bs, D), lambda i: (0, i, 0)),
            pl.BlockSpec((bs, D), lambda i: (i, 0)),
        ],
        out_specs=pl.BlockSpec((B, bs, D), lambda i: (0, i, 0)),
        out_shape=jax.ShapeDtypeStruct((B, S, D), x.dtype),
        compiler_params=pltpu.CompilerParams(
            dimension_semantics=("parallel",),
        ),
    )(x, pos_table)
